# baseline (speedup 1.0000x reference)
import functools

import jax
import jax.numpy as jnp
from jax.experimental import pallas as pl
from jax.experimental.pallas import tpu as pltpu

D = 128
R2 = 16
B = 64
F32 = jnp.float32
_STDC = 0.0010000000474974513
_SCALES = (1.0, 1.5, 2.0 / 3.0)


def _features(m):
    a = jnp.abs(m)
    d = jnp.maximum(2.0 * _STDC - a, 0.0)
    return jnp.concatenate([m, a, d], axis=1)


def _fold(wl):
    x = wl[D:, :].reshape(D, 12, D)
    blocks = []
    for k in range(4):
        acc = None
        for s, sc in enumerate(_SCALES):
            sl = x[:, k * 3 + s, :]
            term = sl * sc if sc != 1.0 else sl
            acc = term if acc is None else acc + term
        blocks.append(acc)
    b0, b1, b2, b3 = blocks
    wm = (b0 + b1 + b2) * 0.5
    wa = (b1 - b2 + b3) * 0.5
    return jnp.concatenate([wm, wa, b3 * 0.5], axis=0), b3


def _rule_kernel(q_ref, wr0_ref, br0_ref, wr1_ref, br1_ref, c_ref,
                 wl0_ref, bl0_ref, wl1_ref, bl1_ref,
                 w1_ref, b1_ref, w2_ref, emb_ref,
                 sub_ref, mask_ref,
                 hid1_s, rel1_s, qw1_s, score_s, wf1_s, beff1_s):
    i = pl.program_id(0)

    @pl.when(i == 0)
    def _layer0():
        q = q_ref[...]
        c = c_ref[...]
        bl0 = bl0_ref[...]
        wf0, w0k3 = _fold(wl0_ref[...])
        wf1, _ = _fold(wl1_ref[...])
        wf1_s[...] = wf1
        h2l0 = jnp.maximum(
            _STDC * jnp.sum(w0k3, axis=0, keepdims=True) + bl0, 0.0)
        beff1_s[...] = (
            jnp.dot(h2l0, wl1_ref[:D, :], preferred_element_type=F32)
            + bl1_ref[...])
        for r in range(R2):
            lo, hi = r * D, (r + 1) * D
            rel1_s[r * B:(r + 1) * B, :] = (
                jnp.dot(q, wr1_ref[:, lo:hi], preferred_element_type=F32)
                + br1_ref[:, lo:hi])
            hid1_s[r * B:(r + 1) * B, :] = c * (
                jnp.dot(q, wr0_ref[:, lo:hi], preferred_element_type=F32)
                + br0_ref[:, lo:hi])
        qw1 = (jnp.dot(q, w1_ref[D:, :], preferred_element_type=F32)
               + b1_ref[...])
        qw1_s[...] = jnp.broadcast_to(qw1[None], (R2, B, D)).reshape(R2 * B, D)
        f1 = _features(hid1_s[...])
        hid1_s[...] = jnp.maximum(
            jnp.dot(f1, wf0, preferred_element_type=F32) + bl0, 0.0)

    h1 = hid1_s[pl.ds(i * B, B), :]
    m2 = (jnp.broadcast_to(h1[None], (R2, B, D)).reshape(R2 * B, D)
          * rel1_s[...])
    f2 = _features(m2)
    hid2 = jnp.maximum(
        jnp.dot(f2, wf1_s[...], preferred_element_type=F32)
        + beff1_s[...], 0.0)
    ho = jnp.maximum(
        jnp.dot(hid2, w1_ref[:D, :], preferred_element_type=F32)
        + qw1_s[...], 0.0)
    sc = jnp.dot(ho, w2_ref[...], preferred_element_type=F32)
    score_s[i] = jnp.concatenate(
        [sc[r * B:(r + 1) * B, :] for r in range(R2)], axis=1)

    @pl.when(i == R2 - 1)
    def _finish():
        s_all = score_s[...]
        mx = jnp.max(jnp.max(s_all, axis=0), axis=1)[None, :, None]
        e = jnp.exp(s_all - mx)
        den = jnp.sum(jnp.sum(e, axis=0), axis=1)[None, :, None]
        att = e / den
        marg0 = jnp.sum(att, axis=2)
        marg1 = jnp.sum(att, axis=0)
        emb = emb_ref[...]
        sub_ref[:, 0, :] = jax.lax.dot_general(
            marg0, emb, (((0,), (0,)), ((), ())), preferred_element_type=F32)
        sub_ref[:, 1, :] = jnp.dot(marg1, emb, preferred_element_type=F32)
        mask_ref[...] = jnp.ones((B, 2), dtype=jnp.bool_)


@functools.partial(jax.jit, static_argnames=("interpret",))
def _run(query, relation_emb, indicator, Wr0, br0, Wl0, bl0,
         Wr1, br1, Wl1, bl1, W1, b1, W2, interpret=False):
    spec = lambda shape: pl.BlockSpec(shape, lambda i: tuple(0 for _ in shape))
    return pl.pallas_call(
        _rule_kernel,
        grid=(R2,),
        in_specs=[
            spec((B, D)),
            spec((D, R2 * D)),
            spec((1, R2 * D)),
            spec((D, R2 * D)),
            spec((1, R2 * D)),
            spec((1, D)),
            spec((13 * D, D)),
            spec((1, D)),
            spec((13 * D, D)),
            spec((1, D)),
            spec((2 * D, D)),
            spec((1, D)),
            spec((D, 1)),
            spec((R2, D)),
        ],
        out_specs=[spec((B, 2, D)), spec((B, 2))],
        out_shape=[jax.ShapeDtypeStruct((B, 2, D), F32),
                   jax.ShapeDtypeStruct((B, 2), jnp.bool_)],
        scratch_shapes=[
            pltpu.VMEM((R2 * B, D), F32),
            pltpu.VMEM((R2 * B, D), F32),
            pltpu.VMEM((R2 * B, D), F32),
            pltpu.VMEM((R2, B, R2), F32),
            pltpu.VMEM((3 * D, D), F32),
            pltpu.VMEM((1, D), F32),
        ],
        interpret=interpret,
    )(query, Wr0, br0.reshape(1, R2 * D), Wr1, br1.reshape(1, R2 * D),
      indicator, Wl0, bl0.reshape(1, D), Wl1, bl1.reshape(1, D),
      W1, b1.reshape(1, D), W2, relation_emb)


def kernel(query, relation_emb, indicator, Wr0, br0, Wl0, bl0,
           Wr1, br1, Wl1, bl1, W1, b1, W2, b2):
    subgoals, masks = _run(query, relation_emb, indicator, Wr0, br0, Wl0,
                           bl0, Wr1, br1, Wl1, bl1, W1, b1, W2)
    return (subgoals, masks)


# speedup vs baseline: 1.0313x; 1.0313x over previous
import functools

import jax
import jax.numpy as jnp
from jax.experimental import pallas as pl
from jax.experimental.pallas import tpu as pltpu

D = 128
R2 = 16
B = 64
F32 = jnp.float32
_STDC = 0.0010000000474974513
_SCALES = (1.0, 1.5, 2.0 / 3.0)


def _features(m):
    a = jnp.abs(m)
    d = jnp.maximum(2.0 * _STDC - a, 0.0)
    return jnp.concatenate([m, a, d], axis=1)


def _fold(wl):
    x = wl[D:, :].reshape(D, 12, D)
    blocks = []
    for k in range(4):
        acc = None
        for s, sc in enumerate(_SCALES):
            sl = x[:, k * 3 + s, :]
            term = sl * sc if sc != 1.0 else sl
            acc = term if acc is None else acc + term
        blocks.append(acc)
    b0, b1, b2, b3 = blocks
    wm = (b0 + b1 + b2) * 0.5
    wa = (b1 - b2 + b3) * 0.5
    return jnp.concatenate([wm, wa, b3 * 0.5], axis=0), b3


def _rule_kernel(q_ref, wr0_ref, br0_ref, wr1_ref, br1_ref, c_ref,
                 wl0_ref, bl0_ref, wl1_ref, bl1_ref,
                 w1_ref, b1_ref, w2_ref, emb_ref,
                 sub_ref,
                 hid1_s, rel1_s, qw1_s, score_s, wf1_s, beff1_s):
    i = pl.program_id(0)

    @pl.when(i == 0)
    def _layer0():
        q = q_ref[...]
        c = c_ref[...]
        bl0 = bl0_ref[...]
        wf0, w0k3 = _fold(wl0_ref[...])
        wf1, _ = _fold(wl1_ref[...])
        wf1_s[...] = wf1
        h2l0 = jnp.maximum(
            _STDC * jnp.sum(w0k3, axis=0, keepdims=True) + bl0, 0.0)
        beff1_s[...] = (
            jnp.dot(h2l0, wl1_ref[:D, :], preferred_element_type=F32)
            + bl1_ref[...])
        for r in range(R2):
            lo, hi = r * D, (r + 1) * D
            rel1_s[r * B:(r + 1) * B, :] = (
                jnp.dot(q, wr1_ref[:, lo:hi], preferred_element_type=F32)
                + br1_ref[:, lo:hi])
            hid1_s[r * B:(r + 1) * B, :] = c * (
                jnp.dot(q, wr0_ref[:, lo:hi], preferred_element_type=F32)
                + br0_ref[:, lo:hi])
        qw1 = (jnp.dot(q, w1_ref[D:, :], preferred_element_type=F32)
               + b1_ref[...])
        qw1_s[...] = jnp.broadcast_to(qw1[None], (R2, B, D)).reshape(R2 * B, D)
        f1 = _features(hid1_s[...])
        hid1_s[...] = jnp.maximum(
            jnp.dot(f1, wf0, preferred_element_type=F32) + bl0, 0.0)

    h1 = hid1_s[pl.ds(i * B, B), :]
    m2 = (jnp.broadcast_to(h1[None], (R2, B, D)).reshape(R2 * B, D)
          * rel1_s[...])
    f2 = _features(m2)
    hid2 = jnp.maximum(
        jnp.dot(f2, wf1_s[...], preferred_element_type=F32)
        + beff1_s[...], 0.0)
    ho = jnp.maximum(
        jnp.dot(hid2, w1_ref[:D, :], preferred_element_type=F32)
        + qw1_s[...], 0.0)
    sc = jnp.dot(ho, w2_ref[...], preferred_element_type=F32)
    score_s[i] = jnp.concatenate(
        [sc[r * B:(r + 1) * B, :] for r in range(R2)], axis=1)

    @pl.when(i == R2 - 1)
    def _finish():
        s_all = score_s[...]
        mx = jnp.max(jnp.max(s_all, axis=0), axis=1)[None, :, None]
        e = jnp.exp(s_all - mx)
        den = jnp.sum(jnp.sum(e, axis=0), axis=1)[None, :, None]
        att = e / den
        marg0 = jnp.sum(att, axis=2)
        marg1 = jnp.sum(att, axis=0)
        emb = emb_ref[...]
        sub_ref[:, 0, :] = jax.lax.dot_general(
            marg0, emb, (((0,), (0,)), ((), ())), preferred_element_type=F32)
        sub_ref[:, 1, :] = jnp.dot(marg1, emb, preferred_element_type=F32)


@functools.partial(jax.jit, static_argnames=("interpret",))
def _run(query, relation_emb, indicator, Wr0, br0, Wl0, bl0,
         Wr1, br1, Wl1, bl1, W1, b1, W2, interpret=False):
    spec = lambda shape: pl.BlockSpec(shape, lambda i: tuple(0 for _ in shape))
    return pl.pallas_call(
        _rule_kernel,
        grid=(R2,),
        in_specs=[
            spec((B, D)),
            spec((D, R2 * D)),
            spec((1, R2 * D)),
            spec((D, R2 * D)),
            spec((1, R2 * D)),
            spec((1, D)),
            spec((13 * D, D)),
            spec((1, D)),
            spec((13 * D, D)),
            spec((1, D)),
            spec((2 * D, D)),
            spec((1, D)),
            spec((D, 1)),
            spec((R2, D)),
        ],
        out_specs=spec((B, 2, D)),
        out_shape=jax.ShapeDtypeStruct((B, 2, D), F32),
        scratch_shapes=[
            pltpu.VMEM((R2 * B, D), F32),
            pltpu.VMEM((R2 * B, D), F32),
            pltpu.VMEM((R2 * B, D), F32),
            pltpu.VMEM((R2, B, R2), F32),
            pltpu.VMEM((3 * D, D), F32),
            pltpu.VMEM((1, D), F32),
        ],
        interpret=interpret,
    )(query, Wr0, br0.reshape(1, R2 * D), Wr1, br1.reshape(1, R2 * D),
      indicator, Wl0, bl0.reshape(1, D), Wl1, bl1.reshape(1, D),
      W1, b1.reshape(1, D), W2, relation_emb)


def kernel(query, relation_emb, indicator, Wr0, br0, Wl0, bl0,
           Wr1, br1, Wl1, bl1, W1, b1, W2, b2):
    subgoals = _run(query, relation_emb, indicator, Wr0, br0, Wl0, bl0,
                    Wr1, br1, Wl1, bl1, W1, b1, W2)
    masks = jnp.ones(subgoals.shape[:-1], dtype=bool)
    return (subgoals, masks)


# speedup vs baseline: 1.1061x; 1.0726x over previous
import functools

import jax
import jax.numpy as jnp
from jax.experimental import pallas as pl
from jax.experimental.pallas import tpu as pltpu

D = 128
R2 = 16
B = 64
F32 = jnp.float32
_STDC = 0.0010000000474974513
_SCALES = (1.0, 1.5, 2.0 / 3.0)


def _features(m):
    a = jnp.abs(m)
    d = jnp.maximum(2.0 * _STDC - a, 0.0)
    return jnp.concatenate([m, a, d], axis=1)


def _fold(wl):
    x = wl[D:, :].reshape(D, 12, D)
    blocks = []
    for k in range(4):
        acc = None
        for s, sc in enumerate(_SCALES):
            sl = x[:, k * 3 + s, :]
            term = sl * sc if sc != 1.0 else sl
            acc = term if acc is None else acc + term
        blocks.append(acc)
    b0, b1, b2, b3 = blocks
    wm = (b0 + b1 + b2) * 0.5
    wa = (b1 - b2 + b3) * 0.5
    return jnp.concatenate([wm, wa, b3 * 0.5], axis=0), b3


def _rule_kernel(q_ref, wr0_ref, br0_ref, wr1_ref, br1_ref, c_ref,
                 wl0_ref, bl0_ref, wl1_ref, bl1_ref,
                 w1_ref, b1_ref, w2_ref, emb_ref,
                 sub_ref,
                 hid1_s, rel1_s, qw1_s, score_s, wf1_s, beff1_s):
    i = pl.program_id(0)

    @pl.when(i == 0)
    def _layer0():
        q = q_ref[...]
        c = c_ref[...]
        bl0 = bl0_ref[...]
        wf0, w0k3 = _fold(wl0_ref[...])
        wf1, _ = _fold(wl1_ref[...])
        wf1_s[...] = wf1
        h2l0 = jnp.maximum(
            _STDC * jnp.sum(w0k3, axis=0, keepdims=True) + bl0, 0.0)
        beff1_s[...] = (
            jnp.dot(h2l0, wl1_ref[:D, :], preferred_element_type=F32)
            + bl1_ref[...])
        for r in range(R2):
            lo, hi = r * D, (r + 1) * D
            rel1_s[r * B:(r + 1) * B, :] = (
                jnp.dot(q, wr1_ref[:, lo:hi], preferred_element_type=F32)
                + br1_ref[:, lo:hi])
            hid1_s[r * B:(r + 1) * B, :] = c * (
                jnp.dot(q, wr0_ref[:, lo:hi], preferred_element_type=F32)
                + br0_ref[:, lo:hi])
        qw1 = (jnp.dot(q, w1_ref[D:, :], preferred_element_type=F32)
               + b1_ref[...])
        qw1_s[...] = jnp.broadcast_to(qw1[None], (R2, B, D)).reshape(R2 * B, D)
        f1 = _features(hid1_s[...])
        hid1_s[...] = jnp.maximum(
            jnp.dot(f1, wf0, preferred_element_type=F32) + bl0, 0.0)

    h1 = hid1_s[pl.ds(i * B, B), :]
    m2 = (jnp.broadcast_to(h1[None], (R2, B, D)).reshape(R2 * B, D)
          * rel1_s[...])
    a2 = jnp.abs(m2)
    d2 = jnp.maximum(2.0 * _STDC - a2, 0.0)
    hid2 = jnp.maximum(
        jnp.dot(m2, wf1_s[:D, :], preferred_element_type=F32)
        + jnp.dot(a2, wf1_s[D:2 * D, :], preferred_element_type=F32)
        + jnp.dot(d2, wf1_s[2 * D:, :], preferred_element_type=F32)
        + beff1_s[...], 0.0)
    ho = jnp.maximum(
        jnp.dot(hid2, w1_ref[:D, :], preferred_element_type=F32)
        + qw1_s[...], 0.0)
    sc = jnp.dot(ho, w2_ref[...], preferred_element_type=F32)
    score_s[i] = jnp.concatenate(
        [sc[r * B:(r + 1) * B, :] for r in range(R2)], axis=1)

    @pl.when(i == R2 - 1)
    def _finish():
        s_all = score_s[...]
        mx = jnp.max(jnp.max(s_all, axis=0), axis=1)[None, :, None]
        e = jnp.exp(s_all - mx)
        den = jnp.sum(jnp.sum(e, axis=0), axis=1)[None, :, None]
        att = e / den
        marg0 = jnp.sum(att, axis=2)
        marg1 = jnp.sum(att, axis=0)
        emb = emb_ref[...]
        sub_ref[:, 0, :] = jax.lax.dot_general(
            marg0, emb, (((0,), (0,)), ((), ())), preferred_element_type=F32)
        sub_ref[:, 1, :] = jnp.dot(marg1, emb, preferred_element_type=F32)


@functools.partial(jax.jit, static_argnames=("interpret",))
def _run(query, relation_emb, indicator, Wr0, br0, Wl0, bl0,
         Wr1, br1, Wl1, bl1, W1, b1, W2, interpret=False):
    spec = lambda shape: pl.BlockSpec(shape, lambda i: tuple(0 for _ in shape))
    return pl.pallas_call(
        _rule_kernel,
        grid=(R2,),
        in_specs=[
            spec((B, D)),
            spec((D, R2 * D)),
            spec((1, R2 * D)),
            spec((D, R2 * D)),
            spec((1, R2 * D)),
            spec((1, D)),
            spec((13 * D, D)),
            spec((1, D)),
            spec((13 * D, D)),
            spec((1, D)),
            spec((2 * D, D)),
            spec((1, D)),
            spec((D, 1)),
            spec((R2, D)),
        ],
        out_specs=spec((B, 2, D)),
        out_shape=jax.ShapeDtypeStruct((B, 2, D), F32),
        scratch_shapes=[
            pltpu.VMEM((R2 * B, D), F32),
            pltpu.VMEM((R2 * B, D), F32),
            pltpu.VMEM((R2 * B, D), F32),
            pltpu.VMEM((R2, B, R2), F32),
            pltpu.VMEM((3 * D, D), F32),
            pltpu.VMEM((1, D), F32),
        ],
        interpret=interpret,
    )(query, Wr0, br0.reshape(1, R2 * D), Wr1, br1.reshape(1, R2 * D),
      indicator, Wl0, bl0.reshape(1, D), Wl1, bl1.reshape(1, D),
      W1, b1.reshape(1, D), W2, relation_emb)


def kernel(query, relation_emb, indicator, Wr0, br0, Wl0, bl0,
           Wr1, br1, Wl1, bl1, W1, b1, W2, b2):
    subgoals = _run(query, relation_emb, indicator, Wr0, br0, Wl0, bl0,
                    Wr1, br1, Wl1, bl1, W1, b1, W2)
    masks = jnp.ones(subgoals.shape[:-1], dtype=bool)
    return (subgoals, masks)


# speedup vs baseline: 1.1742x; 1.0616x over previous
import functools

import jax
import jax.numpy as jnp
from jax.experimental import pallas as pl
from jax.experimental.pallas import tpu as pltpu

D = 128
R2 = 16
B = 64
F32 = jnp.float32
_STDC = 0.0010000000474974513
_SCALES = (1.0, 1.5, 2.0 / 3.0)


def _features(m):
    a = jnp.abs(m)
    d = jnp.maximum(2.0 * _STDC - a, 0.0)
    return jnp.concatenate([m, a, d], axis=1)


def _fold(wl):
    x = wl[D:, :].reshape(D, 12, D)
    blocks = []
    for k in range(4):
        acc = None
        for s, sc in enumerate(_SCALES):
            sl = x[:, k * 3 + s, :]
            term = sl * sc if sc != 1.0 else sl
            acc = term if acc is None else acc + term
        blocks.append(acc)
    b0, b1, b2, b3 = blocks
    wm = (b0 + b1 + b2) * 0.5
    wa = (b1 - b2 + b3) * 0.5
    return jnp.concatenate([wm, wa, b3 * 0.5], axis=0), b3


def _rule_kernel(q_ref, wr0_ref, br0_ref, wr1_ref, br1_ref, c_ref,
                 wl0_ref, bl0_ref, wl1_ref, bl1_ref,
                 w1_ref, b1_ref, w2_ref, emb_ref,
                 sub_ref,
                 hid1_s, rel1_s, qw1_s, score_s, wf1_s, beff1_s):
    i = pl.program_id(0)

    @pl.when(i == 0)
    def _layer0():
        q = q_ref[...]
        c = c_ref[...]
        bl0 = bl0_ref[...]
        wf0, w0k3 = _fold(wl0_ref[...])
        wf1, _ = _fold(wl1_ref[...])
        wf1_s[...] = wf1
        h2l0 = jnp.maximum(
            _STDC * jnp.sum(w0k3, axis=0, keepdims=True) + bl0, 0.0)
        beff1_s[...] = (
            jnp.dot(h2l0, wl1_ref[:D, :], preferred_element_type=F32)
            + bl1_ref[...])
        for r in range(R2):
            lo, hi = r * D, (r + 1) * D
            rel1_s[r * B:(r + 1) * B, :] = (
                jnp.dot(q, wr1_ref[:, lo:hi], preferred_element_type=F32)
                + br1_ref[:, lo:hi])
            hid1_s[r * B:(r + 1) * B, :] = c * (
                jnp.dot(q, wr0_ref[:, lo:hi], preferred_element_type=F32)
                + br0_ref[:, lo:hi])
        qw1 = (jnp.dot(q, w1_ref[D:, :], preferred_element_type=F32)
               + b1_ref[...])
        qw1_s[...] = jnp.broadcast_to(qw1[None], (R2, B, D)).reshape(R2 * B, D)
        m1 = hid1_s[...]
        a1 = jnp.abs(m1)
        d1 = jnp.maximum(2.0 * _STDC - a1, 0.0)
        hid1_s[...] = jnp.maximum(
            jnp.dot(m1, wf0[:D, :], preferred_element_type=F32)
            + jnp.dot(a1, wf0[D:2 * D, :], preferred_element_type=F32)
            + jnp.dot(d1, wf0[2 * D:, :], preferred_element_type=F32)
            + bl0, 0.0)

    for j in range(2):
        h1 = hid1_s[pl.ds((2 * i + j) * B, B), :]
        m2 = (jnp.broadcast_to(h1[None], (R2, B, D)).reshape(R2 * B, D)
              * rel1_s[...])
        a2 = jnp.abs(m2)
        d2 = jnp.maximum(2.0 * _STDC - a2, 0.0)
        hid2 = jnp.maximum(
            jnp.dot(m2, wf1_s[:D, :], preferred_element_type=F32)
            + jnp.dot(a2, wf1_s[D:2 * D, :], preferred_element_type=F32)
            + jnp.dot(d2, wf1_s[2 * D:, :], preferred_element_type=F32)
            + beff1_s[...], 0.0)
        ho = jnp.maximum(
            jnp.dot(hid2, w1_ref[:D, :], preferred_element_type=F32)
            + qw1_s[...], 0.0)
        sc = jnp.dot(ho, w2_ref[...], preferred_element_type=F32)
        score_s[2 * i + j] = jnp.concatenate(
            [sc[r * B:(r + 1) * B, :] for r in range(R2)], axis=1)

    @pl.when(i == R2 // 2 - 1)
    def _finish():
        s_all = score_s[...]
        mx = jnp.max(jnp.max(s_all, axis=0), axis=1)[None, :, None]
        e = jnp.exp(s_all - mx)
        den = jnp.sum(jnp.sum(e, axis=0), axis=1)[None, :, None]
        att = e / den
        marg0 = jnp.sum(att, axis=2)
        marg1 = jnp.sum(att, axis=0)
        emb = emb_ref[...]
        sub_ref[:, 0, :] = jax.lax.dot_general(
            marg0, emb, (((0,), (0,)), ((), ())), preferred_element_type=F32)
        sub_ref[:, 1, :] = jnp.dot(marg1, emb, preferred_element_type=F32)


@functools.partial(jax.jit, static_argnames=("interpret",))
def _run(query, relation_emb, indicator, Wr0, br0, Wl0, bl0,
         Wr1, br1, Wl1, bl1, W1, b1, W2, interpret=False):
    spec = lambda shape: pl.BlockSpec(shape, lambda i: tuple(0 for _ in shape))
    return pl.pallas_call(
        _rule_kernel,
        grid=(R2 // 2,),
        in_specs=[
            spec((B, D)),
            spec((D, R2 * D)),
            spec((1, R2 * D)),
            spec((D, R2 * D)),
            spec((1, R2 * D)),
            spec((1, D)),
            spec((13 * D, D)),
            spec((1, D)),
            spec((13 * D, D)),
            spec((1, D)),
            spec((2 * D, D)),
            spec((1, D)),
            spec((D, 1)),
            spec((R2, D)),
        ],
        out_specs=spec((B, 2, D)),
        out_shape=jax.ShapeDtypeStruct((B, 2, D), F32),
        scratch_shapes=[
            pltpu.VMEM((R2 * B, D), F32),
            pltpu.VMEM((R2 * B, D), F32),
            pltpu.VMEM((R2 * B, D), F32),
            pltpu.VMEM((R2, B, R2), F32),
            pltpu.VMEM((3 * D, D), F32),
            pltpu.VMEM((1, D), F32),
        ],
        interpret=interpret,
    )(query, Wr0, br0.reshape(1, R2 * D), Wr1, br1.reshape(1, R2 * D),
      indicator, Wl0, bl0.reshape(1, D), Wl1, bl1.reshape(1, D),
      W1, b1.reshape(1, D), W2, relation_emb)


def kernel(query, relation_emb, indicator, Wr0, br0, Wl0, bl0,
           Wr1, br1, Wl1, bl1, W1, b1, W2, b2):
    subgoals = _run(query, relation_emb, indicator, Wr0, br0, Wl0, bl0,
                    Wr1, br1, Wl1, bl1, W1, b1, W2)
    masks = jnp.ones(subgoals.shape[:-1], dtype=bool)
    return (subgoals, masks)


# speedup vs baseline: 1.2312x; 1.0485x over previous
import functools

import jax
import jax.numpy as jnp
from jax.experimental import pallas as pl
from jax.experimental.pallas import tpu as pltpu

D = 128
R2 = 16
B = 64
F32 = jnp.float32
_STDC = 0.0010000000474974513
_SCALES = (1.0, 1.5, 2.0 / 3.0)


def _features(m):
    a = jnp.abs(m)
    d = jnp.maximum(2.0 * _STDC - a, 0.0)
    return jnp.concatenate([m, a, d], axis=1)


def _fold(wl):
    x = wl[D:, :].reshape(D, 12, D)
    blocks = []
    for k in range(4):
        acc = None
        for s, sc in enumerate(_SCALES):
            sl = x[:, k * 3 + s, :]
            term = sl * sc if sc != 1.0 else sl
            acc = term if acc is None else acc + term
        blocks.append(acc)
    b0, b1, b2, b3 = blocks
    wm = (b0 + b1 + b2) * 0.5
    wa = (b1 - b2 + b3) * 0.5
    return jnp.concatenate([wm, wa, b3 * 0.5], axis=0), b3


def _rule_kernel(q_ref, wr0_ref, br0_ref, wr1_ref, br1_ref, c_ref,
                 wl0_ref, bl0_ref, wl1_ref, bl1_ref,
                 w1_ref, b1_ref, w2_ref, emb_ref,
                 sub_ref,
                 hid1_s, rel1_s, qw1_s, score_s, wf1_s, beff1_s):
    i = pl.program_id(0)

    @pl.when(i == 0)
    def _layer0():
        q = q_ref[...]
        c = c_ref[...]
        bl0 = bl0_ref[...]
        wf0, w0k3 = _fold(wl0_ref[...])
        wf1, _ = _fold(wl1_ref[...])
        wf1_s[...] = wf1
        h2l0 = jnp.maximum(
            _STDC * jnp.sum(w0k3, axis=0, keepdims=True) + bl0, 0.0)
        beff1_s[...] = (
            jnp.dot(h2l0, wl1_ref[:D, :], preferred_element_type=F32)
            + bl1_ref[...])
        for r in range(R2):
            lo, hi = r * D, (r + 1) * D
            rel1_s[r * B:(r + 1) * B, :] = (
                jnp.dot(q, wr1_ref[:, lo:hi], preferred_element_type=F32)
                + br1_ref[:, lo:hi])
            hid1_s[r * B:(r + 1) * B, :] = c * (
                jnp.dot(q, wr0_ref[:, lo:hi], preferred_element_type=F32)
                + br0_ref[:, lo:hi])
        qw1 = (jnp.dot(q, w1_ref[D:, :], preferred_element_type=F32)
               + b1_ref[...])
        qw1_s[...] = jnp.broadcast_to(qw1[None], (R2, B, D)).reshape(R2 * B, D)
        m1 = hid1_s[...]
        a1 = jnp.abs(m1)
        d1 = jnp.maximum(2.0 * _STDC - a1, 0.0)
        hid1_s[...] = jnp.maximum(
            jnp.dot(m1, wf0[:D, :], preferred_element_type=F32)
            + jnp.dot(a1, wf0[D:2 * D, :], preferred_element_type=F32)
            + jnp.dot(d1, wf0[2 * D:, :], preferred_element_type=F32)
            + bl0, 0.0)

    for j in range(4):
        h1 = hid1_s[pl.ds((4 * i + j) * B, B), :]
        m2 = (jnp.broadcast_to(h1[None], (R2, B, D)).reshape(R2 * B, D)
              * rel1_s[...])
        a2 = jnp.abs(m2)
        d2 = jnp.maximum(2.0 * _STDC - a2, 0.0)
        hid2 = jnp.maximum(
            jnp.dot(m2, wf1_s[:D, :], preferred_element_type=F32)
            + jnp.dot(a2, wf1_s[D:2 * D, :], preferred_element_type=F32)
            + jnp.dot(d2, wf1_s[2 * D:, :], preferred_element_type=F32)
            + beff1_s[...], 0.0)
        ho = jnp.maximum(
            jnp.dot(hid2, w1_ref[:D, :], preferred_element_type=F32)
            + qw1_s[...], 0.0)
        sc = jnp.dot(ho, w2_ref[...], preferred_element_type=F32)
        score_s[4 * i + j] = jnp.concatenate(
            [sc[r * B:(r + 1) * B, :] for r in range(R2)], axis=1)

    @pl.when(i == R2 // 4 - 1)
    def _finish():
        s_all = score_s[...]
        mx = jnp.max(jnp.max(s_all, axis=0), axis=1)[None, :, None]
        e = jnp.exp(s_all - mx)
        den = jnp.sum(jnp.sum(e, axis=0), axis=1)[None, :, None]
        att = e / den
        marg0 = jnp.sum(att, axis=2)
        marg1 = jnp.sum(att, axis=0)
        emb = emb_ref[...]
        sub_ref[:, 0, :] = jax.lax.dot_general(
            marg0, emb, (((0,), (0,)), ((), ())), preferred_element_type=F32)
        sub_ref[:, 1, :] = jnp.dot(marg1, emb, preferred_element_type=F32)


@functools.partial(jax.jit, static_argnames=("interpret",))
def _run(query, relation_emb, indicator, Wr0, br0, Wl0, bl0,
         Wr1, br1, Wl1, bl1, W1, b1, W2, interpret=False):
    spec = lambda shape: pl.BlockSpec(shape, lambda i: tuple(0 for _ in shape))
    return pl.pallas_call(
        _rule_kernel,
        grid=(R2 // 4,),
        in_specs=[
            spec((B, D)),
            spec((D, R2 * D)),
            spec((1, R2 * D)),
            spec((D, R2 * D)),
            spec((1, R2 * D)),
            spec((1, D)),
            spec((13 * D, D)),
            spec((1, D)),
            spec((13 * D, D)),
            spec((1, D)),
            spec((2 * D, D)),
            spec((1, D)),
            spec((D, 1)),
            spec((R2, D)),
        ],
        out_specs=spec((B, 2, D)),
        out_shape=jax.ShapeDtypeStruct((B, 2, D), F32),
        scratch_shapes=[
            pltpu.VMEM((R2 * B, D), F32),
            pltpu.VMEM((R2 * B, D), F32),
            pltpu.VMEM((R2 * B, D), F32),
            pltpu.VMEM((R2, B, R2), F32),
            pltpu.VMEM((3 * D, D), F32),
            pltpu.VMEM((1, D), F32),
        ],
        interpret=interpret,
    )(query, Wr0, br0.reshape(1, R2 * D), Wr1, br1.reshape(1, R2 * D),
      indicator, Wl0, bl0.reshape(1, D), Wl1, bl1.reshape(1, D),
      W1, b1.reshape(1, D), W2, relation_emb)


def kernel(query, relation_emb, indicator, Wr0, br0, Wl0, bl0,
           Wr1, br1, Wl1, bl1, W1, b1, W2, b2):
    subgoals = _run(query, relation_emb, indicator, Wr0, br0, Wl0, bl0,
                    Wr1, br1, Wl1, bl1, W1, b1, W2)
    masks = jnp.ones(subgoals.shape[:-1], dtype=bool)
    return (subgoals, masks)


# speedup vs baseline: 1.2652x; 1.0276x over previous
import functools

import jax
import jax.numpy as jnp
from jax.experimental import pallas as pl
from jax.experimental.pallas import tpu as pltpu

D = 128
R2 = 16
B = 64
F32 = jnp.float32
_STDC = 0.0010000000474974513
_SCALES = (1.0, 1.5, 2.0 / 3.0)


def _features(m):
    a = jnp.abs(m)
    d = jnp.maximum(2.0 * _STDC - a, 0.0)
    return jnp.concatenate([m, a, d], axis=1)


def _fold(wl):
    x = wl[D:, :].reshape(D, 12, D)
    blocks = []
    for k in range(4):
        acc = None
        for s, sc in enumerate(_SCALES):
            sl = x[:, k * 3 + s, :]
            term = sl * sc if sc != 1.0 else sl
            acc = term if acc is None else acc + term
        blocks.append(acc)
    b0, b1, b2, b3 = blocks
    wm = (b0 + b1 + b2) * 0.5
    wa = (b1 - b2 + b3) * 0.5
    return jnp.concatenate([wm, wa, b3 * 0.5], axis=0), b3


def _rule_kernel(q_ref, wr0_ref, br0_ref, wr1_ref, br1_ref, c_ref,
                 wl0_ref, bl0_ref, wl1_ref, bl1_ref,
                 w1_ref, b1_ref, w2_ref, emb_ref,
                 sub_ref,
                 hid1_s, rel1_s, qw1_s, score_s, wf1_s, beff1_s):
    i = pl.program_id(0)

    @pl.when(i == 0)
    def _layer0():
        q = q_ref[...]
        c = c_ref[...]
        bl0 = bl0_ref[...]
        wf0, w0k3 = _fold(wl0_ref[...])
        wf1, _ = _fold(wl1_ref[...])
        wf1_s[...] = wf1
        h2l0 = jnp.maximum(
            _STDC * jnp.sum(w0k3, axis=0, keepdims=True) + bl0, 0.0)
        beff1_s[...] = (
            jnp.dot(h2l0, wl1_ref[:D, :], preferred_element_type=F32)
            + bl1_ref[...])
        for r in range(R2):
            lo, hi = r * D, (r + 1) * D
            rel1_s[r * B:(r + 1) * B, :] = (
                jnp.dot(q, wr1_ref[:, lo:hi], preferred_element_type=F32)
                + br1_ref[:, lo:hi])
            hid1_s[r * B:(r + 1) * B, :] = c * (
                jnp.dot(q, wr0_ref[:, lo:hi], preferred_element_type=F32)
                + br0_ref[:, lo:hi])
        qw1 = (jnp.dot(q, w1_ref[D:, :], preferred_element_type=F32)
               + b1_ref[...])
        qw1_s[...] = jnp.broadcast_to(qw1[None], (R2, B, D)).reshape(R2 * B, D)
        m1 = hid1_s[...]
        a1 = jnp.abs(m1)
        d1 = jnp.maximum(2.0 * _STDC - a1, 0.0)
        hid1_s[...] = jnp.maximum(
            jnp.dot(m1, wf0[:D, :], preferred_element_type=F32)
            + jnp.dot(a1, wf0[D:2 * D, :], preferred_element_type=F32)
            + jnp.dot(d1, wf0[2 * D:, :], preferred_element_type=F32)
            + bl0, 0.0)

    for j in range(8):
        h1 = hid1_s[pl.ds((8 * i + j) * B, B), :]
        m2 = (jnp.broadcast_to(h1[None], (R2, B, D)).reshape(R2 * B, D)
              * rel1_s[...])
        a2 = jnp.abs(m2)
        d2 = jnp.maximum(2.0 * _STDC - a2, 0.0)
        hid2 = jnp.maximum(
            jnp.dot(m2, wf1_s[:D, :], preferred_element_type=F32)
            + jnp.dot(a2, wf1_s[D:2 * D, :], preferred_element_type=F32)
            + jnp.dot(d2, wf1_s[2 * D:, :], preferred_element_type=F32)
            + beff1_s[...], 0.0)
        ho = jnp.maximum(
            jnp.dot(hid2, w1_ref[:D, :], preferred_element_type=F32)
            + qw1_s[...], 0.0)
        sc = jnp.dot(ho, w2_ref[...], preferred_element_type=F32)
        score_s[8 * i + j] = jnp.concatenate(
            [sc[r * B:(r + 1) * B, :] for r in range(R2)], axis=1)

    @pl.when(i == R2 // 8 - 1)
    def _finish():
        s_all = score_s[...]
        mx = jnp.max(jnp.max(s_all, axis=0), axis=1)[None, :, None]
        e = jnp.exp(s_all - mx)
        den = jnp.sum(jnp.sum(e, axis=0), axis=1)[None, :, None]
        att = e / den
        marg0 = jnp.sum(att, axis=2)
        marg1 = jnp.sum(att, axis=0)
        emb = emb_ref[...]
        sub_ref[:, 0, :] = jax.lax.dot_general(
            marg0, emb, (((0,), (0,)), ((), ())), preferred_element_type=F32)
        sub_ref[:, 1, :] = jnp.dot(marg1, emb, preferred_element_type=F32)


@functools.partial(jax.jit, static_argnames=("interpret",))
def _run(query, relation_emb, indicator, Wr0, br0, Wl0, bl0,
         Wr1, br1, Wl1, bl1, W1, b1, W2, interpret=False):
    spec = lambda shape: pl.BlockSpec(shape, lambda i: tuple(0 for _ in shape))
    return pl.pallas_call(
        _rule_kernel,
        grid=(R2 // 8,),
        in_specs=[
            spec((B, D)),
            spec((D, R2 * D)),
            spec((1, R2 * D)),
            spec((D, R2 * D)),
            spec((1, R2 * D)),
            spec((1, D)),
            spec((13 * D, D)),
            spec((1, D)),
            spec((13 * D, D)),
            spec((1, D)),
            spec((2 * D, D)),
            spec((1, D)),
            spec((D, 1)),
            spec((R2, D)),
        ],
        out_specs=spec((B, 2, D)),
        out_shape=jax.ShapeDtypeStruct((B, 2, D), F32),
        scratch_shapes=[
            pltpu.VMEM((R2 * B, D), F32),
            pltpu.VMEM((R2 * B, D), F32),
            pltpu.VMEM((R2 * B, D), F32),
            pltpu.VMEM((R2, B, R2), F32),
            pltpu.VMEM((3 * D, D), F32),
            pltpu.VMEM((1, D), F32),
        ],
        interpret=interpret,
    )(query, Wr0, br0.reshape(1, R2 * D), Wr1, br1.reshape(1, R2 * D),
      indicator, Wl0, bl0.reshape(1, D), Wl1, bl1.reshape(1, D),
      W1, b1.reshape(1, D), W2, relation_emb)


def kernel(query, relation_emb, indicator, Wr0, br0, Wl0, bl0,
           Wr1, br1, Wl1, bl1, W1, b1, W2, b2):
    subgoals = _run(query, relation_emb, indicator, Wr0, br0, Wl0, bl0,
                    Wr1, br1, Wl1, bl1, W1, b1, W2)
    masks = jnp.ones(subgoals.shape[:-1], dtype=bool)
    return (subgoals, masks)


# speedup vs baseline: 1.3485x; 1.0658x over previous
import functools

import jax
import jax.numpy as jnp
from jax.experimental import pallas as pl
from jax.experimental.pallas import tpu as pltpu

D = 128
R2 = 16
B = 64
F32 = jnp.float32
_STDC = 0.0010000000474974513
_SCALES = (1.0, 1.5, 2.0 / 3.0)


def _features(m):
    a = jnp.abs(m)
    d = jnp.maximum(2.0 * _STDC - a, 0.0)
    return jnp.concatenate([m, a, d], axis=1)


def _fold(wl):
    x = wl[D:, :].reshape(D, 12, D)
    blocks = []
    for k in range(4):
        acc = None
        for s, sc in enumerate(_SCALES):
            sl = x[:, k * 3 + s, :]
            term = sl * sc if sc != 1.0 else sl
            acc = term if acc is None else acc + term
        blocks.append(acc)
    b0, b1, b2, b3 = blocks
    wm = (b0 + b1 + b2) * 0.5
    wa = (b1 - b2 + b3) * 0.5
    return jnp.concatenate([wm, wa, b3 * 0.5], axis=0), b3


def _rule_kernel(q_ref, wr0_ref, br0_ref, wr1_ref, br1_ref, c_ref,
                 wl0_ref, bl0_ref, wl1_ref, bl1_ref,
                 w1_ref, b1_ref, w2_ref, emb_ref,
                 sub_ref,
                 hid1_s, rel1_s, qw1_s, score_s, wf1_s, beff1_s):
    i = pl.program_id(0)

    @pl.when(i == 0)
    def _layer0():
        q = q_ref[...]
        c = c_ref[...]
        bl0 = bl0_ref[...]
        wf0, w0k3 = _fold(wl0_ref[...])
        wf1, _ = _fold(wl1_ref[...])
        wf1_s[...] = wf1
        h2l0 = jnp.maximum(
            _STDC * jnp.sum(w0k3, axis=0, keepdims=True) + bl0, 0.0)
        beff1_s[...] = (
            jnp.dot(h2l0, wl1_ref[:D, :], preferred_element_type=F32)
            + bl1_ref[...])
        for r in range(R2):
            lo, hi = r * D, (r + 1) * D
            rel1_s[r * B:(r + 1) * B, :] = (
                jnp.dot(q, wr1_ref[:, lo:hi], preferred_element_type=F32)
                + br1_ref[:, lo:hi])
            hid1_s[r * B:(r + 1) * B, :] = c * (
                jnp.dot(q, wr0_ref[:, lo:hi], preferred_element_type=F32)
                + br0_ref[:, lo:hi])
        qw1 = (jnp.dot(q, w1_ref[D:, :], preferred_element_type=F32)
               + b1_ref[...])
        qw1_s[...] = jnp.broadcast_to(qw1[None], (R2, B, D)).reshape(R2 * B, D)
        m1 = hid1_s[...]
        a1 = jnp.abs(m1)
        d1 = jnp.maximum(2.0 * _STDC - a1, 0.0)
        hid1_s[...] = jnp.maximum(
            jnp.dot(m1, wf0[:D, :], preferred_element_type=F32)
            + jnp.dot(a1, wf0[D:2 * D, :], preferred_element_type=F32)
            + jnp.dot(d1, wf0[2 * D:, :], preferred_element_type=F32)
            + bl0, 0.0)

    for j in range(16):
        h1 = hid1_s[j * B:(j + 1) * B, :]
        m2 = (jnp.broadcast_to(h1[None], (R2, B, D)).reshape(R2 * B, D)
              * rel1_s[...])
        a2 = jnp.abs(m2)
        d2 = jnp.maximum(2.0 * _STDC - a2, 0.0)
        hid2 = jnp.maximum(
            jnp.dot(m2, wf1_s[:D, :], preferred_element_type=F32)
            + jnp.dot(a2, wf1_s[D:2 * D, :], preferred_element_type=F32)
            + jnp.dot(d2, wf1_s[2 * D:, :], preferred_element_type=F32)
            + beff1_s[...], 0.0)
        ho = jnp.maximum(
            jnp.dot(hid2, w1_ref[:D, :], preferred_element_type=F32)
            + qw1_s[...], 0.0)
        sc = jnp.dot(ho, w2_ref[...], preferred_element_type=F32)
        score_s[j] = jnp.concatenate(
            [sc[r * B:(r + 1) * B, :] for r in range(R2)], axis=1)

    @pl.when(i == 0)
    def _finish():
        s_all = score_s[...]
        mx = jnp.max(jnp.max(s_all, axis=0), axis=1)[None, :, None]
        e = jnp.exp(s_all - mx)
        den = jnp.sum(jnp.sum(e, axis=0), axis=1)[None, :, None]
        att = e / den
        marg0 = jnp.sum(att, axis=2)
        marg1 = jnp.sum(att, axis=0)
        emb = emb_ref[...]
        sub_ref[:, 0, :] = jax.lax.dot_general(
            marg0, emb, (((0,), (0,)), ((), ())), preferred_element_type=F32)
        sub_ref[:, 1, :] = jnp.dot(marg1, emb, preferred_element_type=F32)


@functools.partial(jax.jit, static_argnames=("interpret",))
def _run(query, relation_emb, indicator, Wr0, br0, Wl0, bl0,
         Wr1, br1, Wl1, bl1, W1, b1, W2, interpret=False):
    spec = lambda shape: pl.BlockSpec(shape, lambda i: tuple(0 for _ in shape))
    return pl.pallas_call(
        _rule_kernel,
        grid=(1,),
        in_specs=[
            spec((B, D)),
            spec((D, R2 * D)),
            spec((1, R2 * D)),
            spec((D, R2 * D)),
            spec((1, R2 * D)),
            spec((1, D)),
            spec((13 * D, D)),
            spec((1, D)),
            spec((13 * D, D)),
            spec((1, D)),
            spec((2 * D, D)),
            spec((1, D)),
            spec((D, 1)),
            spec((R2, D)),
        ],
        out_specs=spec((B, 2, D)),
        out_shape=jax.ShapeDtypeStruct((B, 2, D), F32),
        scratch_shapes=[
            pltpu.VMEM((R2 * B, D), F32),
            pltpu.VMEM((R2 * B, D), F32),
            pltpu.VMEM((R2 * B, D), F32),
            pltpu.VMEM((R2, B, R2), F32),
            pltpu.VMEM((3 * D, D), F32),
            pltpu.VMEM((1, D), F32),
        ],
        interpret=interpret,
    )(query, Wr0, br0.reshape(1, R2 * D), Wr1, br1.reshape(1, R2 * D),
      indicator, Wl0, bl0.reshape(1, D), Wl1, bl1.reshape(1, D),
      W1, b1.reshape(1, D), W2, relation_emb)


def kernel(query, relation_emb, indicator, Wr0, br0, Wl0, bl0,
           Wr1, br1, Wl1, bl1, W1, b1, W2, b2):
    subgoals = _run(query, relation_emb, indicator, Wr0, br0, Wl0, bl0,
                    Wr1, br1, Wl1, bl1, W1, b1, W2)
    masks = jnp.ones(subgoals.shape[:-1], dtype=bool)
    return (subgoals, masks)


# speedup vs baseline: 1.3536x; 1.0038x over previous
import functools

import jax
import jax.numpy as jnp
from jax.experimental import pallas as pl
from jax.experimental.pallas import tpu as pltpu

D = 128
R2 = 16
B = 64
F32 = jnp.float32
_STDC = 0.0010000000474974513
_SCALES = (1.0, 1.5, 2.0 / 3.0)


def _features(m):
    a = jnp.abs(m)
    d = jnp.maximum(2.0 * _STDC - a, 0.0)
    return jnp.concatenate([m, a, d], axis=1)


def _fold(wl):
    x = wl[D:, :].reshape(D, 12, D)
    blocks = []
    for k in range(4):
        acc = None
        for s, sc in enumerate(_SCALES):
            sl = x[:, k * 3 + s, :]
            term = sl * sc if sc != 1.0 else sl
            acc = term if acc is None else acc + term
        blocks.append(acc)
    b0, b1, b2, b3 = blocks
    wm = (b0 + b1 + b2) * 0.5
    wa = (b1 - b2 + b3) * 0.5
    return jnp.concatenate([wm, wa, b3 * 0.5], axis=0), b3


def _rule_kernel(q_ref, wr0_ref, br0_ref, wr1_ref, br1_ref, c_ref,
                 wl0_ref, bl0_ref, wl1_ref, bl1_ref,
                 w1_ref, b1_ref, w2_ref, emb_ref,
                 sub_ref,
                 hid1_s, rel1_s, qw1_s, score_s, wf1_s, beff1_s):
    if True:
        q = q_ref[...]
        c = c_ref[...]
        bl0 = bl0_ref[...]
        wf0, w0k3 = _fold(wl0_ref[...])
        wf1, _ = _fold(wl1_ref[...])
        wf1_s[...] = wf1
        h2l0 = jnp.maximum(
            _STDC * jnp.sum(w0k3, axis=0, keepdims=True) + bl0, 0.0)
        beff1_s[...] = (
            jnp.dot(h2l0, wl1_ref[:D, :], preferred_element_type=F32)
            + bl1_ref[...])
        for r in range(R2):
            lo, hi = r * D, (r + 1) * D
            rel1_s[r * B:(r + 1) * B, :] = (
                jnp.dot(q, wr1_ref[:, lo:hi], preferred_element_type=F32)
                + br1_ref[:, lo:hi])
            hid1_s[r * B:(r + 1) * B, :] = c * (
                jnp.dot(q, wr0_ref[:, lo:hi], preferred_element_type=F32)
                + br0_ref[:, lo:hi])
        qw1 = (jnp.dot(q, w1_ref[D:, :], preferred_element_type=F32)
               + b1_ref[...])
        qw1_s[...] = jnp.broadcast_to(qw1[None], (R2, B, D)).reshape(R2 * B, D)
        m1 = hid1_s[...]
        a1 = jnp.abs(m1)
        d1 = jnp.maximum(2.0 * _STDC - a1, 0.0)
        hid1_s[...] = jnp.maximum(
            jnp.dot(m1, wf0[:D, :], preferred_element_type=F32)
            + jnp.dot(a1, wf0[D:2 * D, :], preferred_element_type=F32)
            + jnp.dot(d1, wf0[2 * D:, :], preferred_element_type=F32)
            + bl0, 0.0)

    for j in range(16):
        h1 = hid1_s[j * B:(j + 1) * B, :]
        m2 = (jnp.broadcast_to(h1[None], (R2, B, D)).reshape(R2 * B, D)
              * rel1_s[...])
        a2 = jnp.abs(m2)
        d2 = jnp.maximum(2.0 * _STDC - a2, 0.0)
        hid2 = jnp.maximum(
            jnp.dot(m2, wf1_s[:D, :], preferred_element_type=F32)
            + jnp.dot(a2, wf1_s[D:2 * D, :], preferred_element_type=F32)
            + jnp.dot(d2, wf1_s[2 * D:, :], preferred_element_type=F32)
            + beff1_s[...], 0.0)
        ho = jnp.maximum(
            jnp.dot(hid2, w1_ref[:D, :], preferred_element_type=F32)
            + qw1_s[...], 0.0)
        sc = jnp.dot(ho, w2_ref[...], preferred_element_type=F32)
        score_s[j] = jnp.concatenate(
            [sc[r * B:(r + 1) * B, :] for r in range(R2)], axis=1)

    if True:
        s_all = score_s[...]
        mx = jnp.max(jnp.max(s_all, axis=0), axis=1)[None, :, None]
        e = jnp.exp(s_all - mx)
        den = jnp.sum(jnp.sum(e, axis=0), axis=1)[None, :, None]
        att = e / den
        marg0 = jnp.sum(att, axis=2)
        marg1 = jnp.sum(att, axis=0)
        emb = emb_ref[...]
        sub_ref[:, 0, :] = jax.lax.dot_general(
            marg0, emb, (((0,), (0,)), ((), ())), preferred_element_type=F32)
        sub_ref[:, 1, :] = jnp.dot(marg1, emb, preferred_element_type=F32)


@functools.partial(jax.jit, static_argnames=("interpret",))
def _run(query, relation_emb, indicator, Wr0, br0, Wl0, bl0,
         Wr1, br1, Wl1, bl1, W1, b1, W2, interpret=False):
    spec = lambda shape: pl.BlockSpec(shape, lambda i: tuple(0 for _ in shape))
    return pl.pallas_call(
        _rule_kernel,
        grid=(1,),
        in_specs=[
            spec((B, D)),
            spec((D, R2 * D)),
            spec((1, R2 * D)),
            spec((D, R2 * D)),
            spec((1, R2 * D)),
            spec((1, D)),
            spec((13 * D, D)),
            spec((1, D)),
            spec((13 * D, D)),
            spec((1, D)),
            spec((2 * D, D)),
            spec((1, D)),
            spec((D, 1)),
            spec((R2, D)),
        ],
        out_specs=spec((B, 2, D)),
        out_shape=jax.ShapeDtypeStruct((B, 2, D), F32),
        scratch_shapes=[
            pltpu.VMEM((R2 * B, D), F32),
            pltpu.VMEM((R2 * B, D), F32),
            pltpu.VMEM((R2 * B, D), F32),
            pltpu.VMEM((R2, B, R2), F32),
            pltpu.VMEM((3 * D, D), F32),
            pltpu.VMEM((1, D), F32),
        ],
        interpret=interpret,
    )(query, Wr0, br0.reshape(1, R2 * D), Wr1, br1.reshape(1, R2 * D),
      indicator, Wl0, bl0.reshape(1, D), Wl1, bl1.reshape(1, D),
      W1, b1.reshape(1, D), W2, relation_emb)


def kernel(query, relation_emb, indicator, Wr0, br0, Wl0, bl0,
           Wr1, br1, Wl1, bl1, W1, b1, W2, b2):
    subgoals = _run(query, relation_emb, indicator, Wr0, br0, Wl0, bl0,
                    Wr1, br1, Wl1, bl1, W1, b1, W2)
    masks = jnp.ones(subgoals.shape[:-1], dtype=bool)
    return (subgoals, masks)


# speedup vs baseline: 1.3561x; 1.0018x over previous
import functools

import jax
import jax.numpy as jnp
from jax.experimental import pallas as pl
from jax.experimental.pallas import tpu as pltpu

D = 128
R2 = 16
B = 64
F32 = jnp.float32
_STDC = 0.0010000000474974513
_SCALES = (1.0, 1.5, 2.0 / 3.0)


def _fold(wl):
    x = wl[D:, :].reshape(D, 12, D)
    blocks = []
    for k in range(4):
        acc = None
        for s, sc in enumerate(_SCALES):
            sl = x[:, k * 3 + s, :]
            term = sl * sc if sc != 1.0 else sl
            acc = term if acc is None else acc + term
        blocks.append(acc)
    b0, b1, b2, b3 = blocks
    wm = (b0 + b1 + b2) * 0.5
    wa = (b1 - b2 + b3) * 0.5
    return jnp.concatenate([wm, wa, b3 * 0.5], axis=0), b3


def _rule_kernel(q_ref, wr0_ref, br0_ref, wr1_ref, br1_ref, c_ref,
                 wl0_ref, bl0_ref, wl1_ref, bl1_ref,
                 w1_ref, b1_ref, w2_ref, emb_ref,
                 sub_ref,
                 hid1_s, rel1_s, qw1_s, score_s, wf1_s, beff1_s):
    q = q_ref[...]
    c = c_ref[...]
    bl0 = bl0_ref[...]
    wf0, w0k3 = _fold(wl0_ref[...])
    wf1, _ = _fold(wl1_ref[...])
    wf1_s[...] = wf1
    h2l0 = jnp.maximum(
        _STDC * jnp.sum(w0k3, axis=0, keepdims=True) + bl0, 0.0)
    beff1_s[...] = (
        jnp.dot(h2l0, wl1_ref[:D, :], preferred_element_type=F32)
        + bl1_ref[...])
    for r in range(R2):
        lo, hi = r * D, (r + 1) * D
        rel1_s[r * B:(r + 1) * B, :] = (
            jnp.dot(q, wr1_ref[:, lo:hi], preferred_element_type=F32)
            + br1_ref[:, lo:hi])
        hid1_s[r * B:(r + 1) * B, :] = c * (
            jnp.dot(q, wr0_ref[:, lo:hi], preferred_element_type=F32)
            + br0_ref[:, lo:hi])
    qw1 = (jnp.dot(q, w1_ref[D:, :], preferred_element_type=F32)
           + b1_ref[...])
    qw1_s[...] = jnp.broadcast_to(qw1[None], (R2, B, D)).reshape(R2 * B, D)
    m1 = hid1_s[...]
    a1 = jnp.abs(m1)
    d1 = jnp.maximum(2.0 * _STDC - a1, 0.0)
    hid1_s[...] = jnp.maximum(
        jnp.dot(m1, wf0[:D, :], preferred_element_type=F32)
        + jnp.dot(a1, wf0[D:2 * D, :], preferred_element_type=F32)
        + jnp.dot(d1, wf0[2 * D:, :], preferred_element_type=F32)
        + bl0, 0.0)

    for j in range(R2):
        h1 = hid1_s[j * B:(j + 1) * B, :]
        m2 = (jnp.broadcast_to(h1[None], (R2, B, D)).reshape(R2 * B, D)
              * rel1_s[...])
        a2 = jnp.abs(m2)
        d2 = jnp.maximum(2.0 * _STDC - a2, 0.0)
        hid2 = jnp.maximum(
            jnp.dot(m2, wf1_s[:D, :], preferred_element_type=F32)
            + jnp.dot(a2, wf1_s[D:2 * D, :], preferred_element_type=F32)
            + jnp.dot(d2, wf1_s[2 * D:, :], preferred_element_type=F32)
            + beff1_s[...], 0.0)
        ho = jnp.maximum(
            jnp.dot(hid2, w1_ref[:D, :], preferred_element_type=F32)
            + qw1_s[...], 0.0)
        sc = jnp.dot(ho, w2_ref[...], preferred_element_type=F32)
        score_s[j] = jnp.concatenate(
            [sc[r * B:(r + 1) * B, :] for r in range(R2)], axis=1)

    s_all = score_s[...]
    mx = jnp.max(jnp.max(s_all, axis=0), axis=1)[None, :, None]
    e = jnp.exp(s_all - mx)
    den = jnp.sum(jnp.sum(e, axis=0), axis=1)[None, :, None]
    att = e / den
    marg0 = jnp.sum(att, axis=2)
    marg1 = jnp.sum(att, axis=0)
    emb = emb_ref[...]
    sub_ref[:, 0, :] = jax.lax.dot_general(
        marg0, emb, (((0,), (0,)), ((), ())), preferred_element_type=F32)
    sub_ref[:, 1, :] = jnp.dot(marg1, emb, preferred_element_type=F32)


@functools.partial(jax.jit, static_argnames=("interpret",))
def _run(query, relation_emb, indicator, Wr0, br0, Wl0, bl0,
         Wr1, br1, Wl1, bl1, W1, b1, W2, interpret=False):
    spec = lambda shape: pl.BlockSpec(shape, lambda i: tuple(0 for _ in shape))
    return pl.pallas_call(
        _rule_kernel,
        grid=(1,),
        in_specs=[
            spec((B, D)),
            spec((D, R2 * D)),
            spec((1, R2 * D)),
            spec((D, R2 * D)),
            spec((1, R2 * D)),
            spec((1, D)),
            spec((13 * D, D)),
            spec((1, D)),
            spec((13 * D, D)),
            spec((1, D)),
            spec((2 * D, D)),
            spec((1, D)),
            spec((D, 1)),
            spec((R2, D)),
        ],
        out_specs=spec((B, 2, D)),
        out_shape=jax.ShapeDtypeStruct((B, 2, D), F32),
        scratch_shapes=[
            pltpu.VMEM((R2 * B, D), F32),
            pltpu.VMEM((R2 * B, D), F32),
            pltpu.VMEM((R2 * B, D), F32),
            pltpu.VMEM((R2, B, R2), F32),
            pltpu.VMEM((3 * D, D), F32),
            pltpu.VMEM((1, D), F32),
        ],
        interpret=interpret,
    )(query, Wr0, br0.reshape(1, R2 * D), Wr1, br1.reshape(1, R2 * D),
      indicator, Wl0, bl0.reshape(1, D), Wl1, bl1.reshape(1, D),
      W1, b1.reshape(1, D), W2, relation_emb)


def kernel(query, relation_emb, indicator, Wr0, br0, Wl0, bl0,
           Wr1, br1, Wl1, bl1, W1, b1, W2, b2):
    subgoals = _run(query, relation_emb, indicator, Wr0, br0, Wl0, bl0,
                    Wr1, br1, Wl1, bl1, W1, b1, W2)
    masks = jnp.ones(subgoals.shape[:-1], dtype=bool)
    return (subgoals, masks)
